# baseline (speedup 1.0000x reference)
import jax
import jax.numpy as jnp
from jax import lax
from jax.experimental import pallas as pl
from jax.experimental.pallas import tpu as pltpu
from jax.experimental.pallas import tpu_sc as plsc

N_CORES = 2
N_SUBCORES = 16
LANES = 16
N_WORKERS = N_CORES * N_SUBCORES

CHUNK = 2000
N_NODES = 50000


def _edge_body(px, py, pz, row_hbm, col_hbm, shx_hbm, shy_hbm, shz_hbm,
               dist_hbm, vx_hbm, vy_hbm, vz_hbm,
               idxb0, idxb1, gx0, gx1, gy0, gy1, gz0, gz1,
               shx0, shx1, shy0, shy1, shz0, shz1,
               od0, od1, ovx0, ovx1, ovy0, ovy1, ovz0, ovz1,
               sx, sy, sz, semL, semG0, semG1, semO0, semO1):
    n_edges = row_hbm.shape[0]
    per_worker = n_edges // N_WORKERS
    n_chunks = per_worker // CHUNK
    sid = lax.axis_index("s")
    wid = sid * N_CORES + lax.axis_index("c")
    wbase = wid * per_worker

    idxb = (idxb0, idxb1)
    gx, gy, gz = (gx0, gx1), (gy0, gy1), (gz0, gz1)
    shx, shy, shz = (shx0, shx1), (shy0, shy1), (shz0, shz1)
    od, ovx, ovy, ovz = (od0, od1), (ovx0, ovx1), (ovy0, ovy1), (ovz0, ovz1)
    semG = (semG0, semG1)
    semO = (semO0, semO1)

    @pl.when(sid == 0)
    def _stage():
        pltpu.sync_copy(px, sx)
        pltpu.sync_copy(py, sy)
        pltpu.sync_copy(pz, sz)

    plsc.subcore_barrier()

    def issue_linear(k):
        s = k % 2
        sl = pl.ds(wbase + k * CHUNK, CHUNK)
        return [
            pltpu.async_copy(row_hbm.at[sl], idxb[s].at[pl.ds(0, CHUNK)], semL),
            pltpu.async_copy(col_hbm.at[sl], idxb[s].at[pl.ds(CHUNK, CHUNK)], semL),
            pltpu.async_copy(shx_hbm.at[sl], shx[s], semL),
            pltpu.async_copy(shy_hbm.at[sl], shy[s], semL),
            pltpu.async_copy(shz_hbm.at[sl], shz[s], semL),
        ]

    def issue_gathers(k):
        s = k % 2
        return [
            pltpu.async_copy(sx.at[idxb[s]], gx[s], semG[s]),
            pltpu.async_copy(sy.at[idxb[s]], gy[s], semG[s]),
            pltpu.async_copy(sz.at[idxb[s]], gz[s], semG[s]),
        ]

    def issue_out(k):
        s = k % 2
        sl = pl.ds(wbase + k * CHUNK, CHUNK)
        return [
            pltpu.async_copy(od[s], dist_hbm.at[sl], semO[s]),
            pltpu.async_copy(ovx[s], vx_hbm.at[sl], semO[s]),
            pltpu.async_copy(ovy[s], vy_hbm.at[sl], semO[s]),
            pltpu.async_copy(ovz[s], vz_hbm.at[sl], semO[s]),
        ]

    def compute(k):
        s = k % 2
        cgx, cgy, cgz = gx[s], gy[s], gz[s]
        csx, csy, csz = shx[s], shy[s], shz[s]
        cod, cvx, cvy, cvz = od[s], ovx[s], ovy[s], ovz[s]
        cidx = idxb[s]

        @plsc.parallel_loop(0, CHUNK // LANES, 1, unroll=4)
        def vec_body(j):
            e16 = j * LANES
            v = pl.ds(e16, LANES)
            vc = pl.ds(CHUNK + e16, LANES)
            m = cidx[v] != cidx[vc]
            d0 = cgx[v] - cgx[vc] - csx[v]
            d1 = cgy[v] - cgy[vc] - csy[v]
            d2 = cgz[v] - cgz[vc] - csz[v]
            sq = d0 * d0 + d1 * d1 + d2 * d2
            bits = plsc.bitcast(sq, jnp.int32)
            y = plsc.bitcast(jnp.int32(0x5F3759DF) - (bits >> 1), jnp.float32)
            for _ in range(3):
                y = y * (1.5 - 0.5 * sq * y * y)
            cod[v] = jnp.where(m, sq * y, 0.0)
            ym = jnp.where(m, y, 1.0)
            cvx[v] = d0 * ym
            cvy[v] = d1 * ym
            cvz[v] = d2 * ym

    hL, hG, hO = {}, {}, {}
    hL[0] = issue_linear(0)
    for h in hL[0]:
        h.wait()
    hG[0] = issue_gathers(0)
    if n_chunks > 1:
        hL[1] = issue_linear(1)
    for k in range(n_chunks):
        if k + 1 < n_chunks:
            for h in hL[k + 1]:
                h.wait()
            hG[k + 1] = issue_gathers(k + 1)
        for h in hG[k]:
            h.wait()
        if k >= 2:
            for h in hO[k - 2]:
                h.wait()
        compute(k)
        hO[k] = issue_out(k)
        if k + 2 < n_chunks:
            hL[k + 2] = issue_linear(k + 2)
    for k in (n_chunks - 2, n_chunks - 1):
        for h in hO[k]:
            h.wait()


def kernel(positions, edge_indices, shift):
    px = jnp.asarray(positions[:, 0])
    py = jnp.asarray(positions[:, 1])
    pz = jnp.asarray(positions[:, 2])
    row = edge_indices[0]
    col = edge_indices[1]
    shx = shift[:, 0]
    shy = shift[:, 1]
    shz = shift[:, 2]
    n_edges = row.shape[0]
    mesh = plsc.VectorSubcoreMesh(core_axis_name="c", subcore_axis_name="s")
    f = pltpu.VMEM((CHUNK,), jnp.float32)
    f2 = pltpu.VMEM((2 * CHUNK,), jnp.float32)
    i2 = pltpu.VMEM((2 * CHUNK,), jnp.int32)
    fn = pl.kernel(
        _edge_body,
        out_type=(jax.ShapeDtypeStruct((n_edges,), jnp.float32),) * 4,
        mesh=mesh,
        compiler_params=pltpu.CompilerParams(needs_layout_passes=False),
        scratch_types=[
            i2, i2,
            f2, f2, f2, f2, f2, f2,
            f, f, f, f, f, f,
            f, f, f, f, f, f, f, f,
            pltpu.VMEM_SHARED((N_NODES,), jnp.float32),
            pltpu.VMEM_SHARED((N_NODES,), jnp.float32),
            pltpu.VMEM_SHARED((N_NODES,), jnp.float32),
            pltpu.SemaphoreType.DMA,
            pltpu.SemaphoreType.DMA,
            pltpu.SemaphoreType.DMA,
            pltpu.SemaphoreType.DMA,
            pltpu.SemaphoreType.DMA,
        ],
    )
    dist, vx, vy, vz = fn(px, py, pz, row, col, shx, shy, shz)
    vec = jnp.stack([vx, vy, vz], axis=0).T
    return (edge_indices, dist, vec)


# speedup vs baseline: 1.0796x; 1.0796x over previous
import jax
import jax.numpy as jnp
from jax import lax
from jax.experimental import pallas as pl
from jax.experimental.pallas import tpu as pltpu
from jax.experimental.pallas import tpu_sc as plsc

N_CORES = 2
N_SUBCORES = 16
LANES = 16
N_WORKERS = N_CORES * N_SUBCORES

CHUNK = 2000
N_NODES = 50000
SLICE_B = CHUNK // 128 + 2
SLICE_W = SLICE_B * 256


def _edge_body(px, py, pz, el_hbm, shx_hbm, shy_hbm, shz_hbm,
               dist_hbm, vx_hbm, vy_hbm, vz_hbm,
               idxb0, idxb1, gx0, gx1, gy0, gy1, gz0, gz1,
               shx0, shx1, shy0, shy1, shz0, shz1,
               od0, od1, ovx0, ovx1, ovy0, ovy1, ovz0, ovz1,
               sx, sy, sz, semL, semG0, semG1, semO0, semO1):
    n_edges = shx_hbm.shape[0]
    per_worker = n_edges // N_WORKERS
    n_chunks = per_worker // CHUNK
    n_blocks = n_edges // 128
    sid = lax.axis_index("s")
    wid = sid * N_CORES + lax.axis_index("c")
    wbase = wid * per_worker

    def slice_start(k):
        base = wbase + k * CHUNK
        return jnp.minimum(base >> 7, n_blocks - SLICE_B)

    idxb = (idxb0, idxb1)
    gx, gy, gz = (gx0, gx1), (gy0, gy1), (gz0, gz1)
    shx, shy, shz = (shx0, shx1), (shy0, shy1), (shz0, shz1)
    od, ovx, ovy, ovz = (od0, od1), (ovx0, ovx1), (ovy0, ovy1), (ovz0, ovz1)
    semG = (semG0, semG1)
    semO = (semO0, semO1)

    @pl.when(sid == 0)
    def _stage():
        pltpu.sync_copy(px, sx)
        pltpu.sync_copy(py, sy)
        pltpu.sync_copy(pz, sz)

    plsc.subcore_barrier()

    def issue_linear(k):
        s = k % 2
        sl = pl.ds(wbase + k * CHUNK, CHUNK)
        return [
            pltpu.async_copy(el_hbm.at[pl.ds(slice_start(k) * 256, SLICE_W)],
                             idxb[s], semL),
            pltpu.async_copy(shx_hbm.at[sl], shx[s], semL),
            pltpu.async_copy(shy_hbm.at[sl], shy[s], semL),
            pltpu.async_copy(shz_hbm.at[sl], shz[s], semL),
        ]

    def issue_gathers(k):
        s = k % 2
        return [
            pltpu.async_copy(sx.at[idxb[s]], gx[s], semG[s]),
            pltpu.async_copy(sy.at[idxb[s]], gy[s], semG[s]),
            pltpu.async_copy(sz.at[idxb[s]], gz[s], semG[s]),
        ]

    def issue_out(k):
        s = k % 2
        sl = pl.ds(wbase + k * CHUNK, CHUNK)
        return [
            pltpu.async_copy(od[s], dist_hbm.at[sl], semO[s]),
            pltpu.async_copy(ovx[s], vx_hbm.at[sl], semO[s]),
            pltpu.async_copy(ovy[s], vy_hbm.at[sl], semO[s]),
            pltpu.async_copy(ovz[s], vz_hbm.at[sl], semO[s]),
        ]

    def compute(k):
        s = k % 2
        cgx, cgy, cgz = gx[s], gy[s], gz[s]
        csx, csy, csz = shx[s], shy[s], shz[s]
        cod, cvx, cvy, cvz = od[s], ovx[s], ovy[s], ovz[s]
        cidx = idxb[s]
        a0 = wbase + k * CHUNK - (slice_start(k) << 7)

        @plsc.parallel_loop(0, CHUNK // LANES, 1, unroll=2)
        def vec_body(j):
            e16 = j * LANES
            v = pl.ds(e16, LANES)
            e = a0 + e16
            ro = e + ((e >> 7) << 7)
            vr = pl.ds(ro, LANES)
            vc = pl.ds(ro + 128, LANES)
            m = cidx[vr] != cidx[vc]
            d0 = cgx[vr] - cgx[vc] - csx[v]
            d1 = cgy[vr] - cgy[vc] - csy[v]
            d2 = cgz[vr] - cgz[vc] - csz[v]
            sq = d0 * d0 + d1 * d1 + d2 * d2
            bits = plsc.bitcast(sq, jnp.int32)
            y = plsc.bitcast(jnp.int32(0x5F3759DF) - (bits >> 1), jnp.float32)
            for _ in range(3):
                y = y * (1.5 - 0.5 * sq * y * y)
            cod[v] = jnp.where(m, sq * y, 0.0)
            ym = jnp.where(m, y, 1.0)
            cvx[v] = d0 * ym
            cvy[v] = d1 * ym
            cvz[v] = d2 * ym

    hL, hG, hO = {}, {}, {}
    hL[0] = issue_linear(0)
    for h in hL[0]:
        h.wait()
    hG[0] = issue_gathers(0)
    if n_chunks > 1:
        hL[1] = issue_linear(1)
    for k in range(n_chunks):
        if k + 1 < n_chunks:
            for h in hL[k + 1]:
                h.wait()
            hG[k + 1] = issue_gathers(k + 1)
        for h in hG[k]:
            h.wait()
        if k >= 2:
            for h in hO[k - 2]:
                h.wait()
        compute(k)
        hO[k] = issue_out(k)
        if k + 2 < n_chunks:
            hL[k + 2] = issue_linear(k + 2)
    for k in (n_chunks - 2, n_chunks - 1):
        for h in hO[k]:
            h.wait()


def kernel(positions, edge_indices, shift):
    px = jnp.asarray(positions[:, 0])
    py = jnp.asarray(positions[:, 1])
    pz = jnp.asarray(positions[:, 2])
    n_blocks = edge_indices.shape[1] // 128
    el = edge_indices.reshape(2, n_blocks, 128).transpose(1, 0, 2).reshape(-1)
    shx = shift[:, 0]
    shy = shift[:, 1]
    shz = shift[:, 2]
    n_edges = edge_indices.shape[1]
    mesh = plsc.VectorSubcoreMesh(core_axis_name="c", subcore_axis_name="s")
    f = pltpu.VMEM((CHUNK,), jnp.float32)
    f2 = pltpu.VMEM((SLICE_W,), jnp.float32)
    i2 = pltpu.VMEM((SLICE_W,), jnp.int32)
    fn = pl.kernel(
        _edge_body,
        out_type=(jax.ShapeDtypeStruct((n_edges,), jnp.float32),) * 4,
        mesh=mesh,
        compiler_params=pltpu.CompilerParams(needs_layout_passes=False),
        scratch_types=[
            i2, i2,
            f2, f2, f2, f2, f2, f2,
            f, f, f, f, f, f,
            f, f, f, f, f, f, f, f,
            pltpu.VMEM_SHARED((N_NODES,), jnp.float32),
            pltpu.VMEM_SHARED((N_NODES,), jnp.float32),
            pltpu.VMEM_SHARED((N_NODES,), jnp.float32),
            pltpu.SemaphoreType.DMA,
            pltpu.SemaphoreType.DMA,
            pltpu.SemaphoreType.DMA,
            pltpu.SemaphoreType.DMA,
            pltpu.SemaphoreType.DMA,
        ],
    )
    dist, vx, vy, vz = fn(px, py, pz, el, shx, shy, shz)
    vec = jnp.stack([vx, vy, vz], axis=0).T
    return (edge_indices, dist, vec)


# speedup vs baseline: 1.2461x; 1.1542x over previous
import jax
import jax.numpy as jnp
from jax import lax
from jax.experimental import pallas as pl
from jax.experimental.pallas import tpu as pltpu
from jax.experimental.pallas import tpu_sc as plsc

N_CORES = 2
N_SUBCORES = 16
LANES = 16
N_WORKERS = N_CORES * N_SUBCORES

CHUNK = 2000
N_NODES = 50000
SLICE_B = CHUNK // 128 + 2
SLICE_W = SLICE_B * 256


def _edge_body(px, py, pz, el_hbm, sh3_hbm,
               dist_hbm, vx_hbm, vy_hbm, vz_hbm,
               idxb0, idxb1, gx0, gx1, gy0, gy1, gz0, gz1,
               shb0, shb1,
               od0, od1, ovx0, ovx1, ovy0, ovy1, ovz0, ovz1,
               sx, sy, sz, semL, semG0, semG1, semO0, semO1):
    n_edges = dist_hbm.shape[0]
    per_worker = n_edges // N_WORKERS
    n_chunks = per_worker // CHUNK
    n_blocks = n_edges // 128
    sid = lax.axis_index("s")
    wid = sid * N_CORES + lax.axis_index("c")
    wbase = wid * per_worker

    def slice_start(k):
        base = wbase + k * CHUNK
        return jnp.minimum(base >> 7, n_blocks - SLICE_B)

    idxb = (idxb0, idxb1)
    gx, gy, gz = (gx0, gx1), (gy0, gy1), (gz0, gz1)
    shb = (shb0, shb1)
    od, ovx, ovy, ovz = (od0, od1), (ovx0, ovx1), (ovy0, ovy1), (ovz0, ovz1)
    semG = (semG0, semG1)
    semO = (semO0, semO1)

    @pl.when(sid == 0)
    def _stage():
        pltpu.sync_copy(px, sx)
        pltpu.sync_copy(py, sy)
        pltpu.sync_copy(pz, sz)

    plsc.subcore_barrier()

    def issue_linear(k):
        s = k % 2
        s0 = slice_start(k)
        return [
            pltpu.async_copy(el_hbm.at[pl.ds(s0 * 256, SLICE_W)],
                             idxb[s], semL),
            pltpu.async_copy(sh3_hbm.at[pl.ds(s0 * 384, SLICE_B * 384)],
                             shb[s], semL),
        ]

    def issue_gathers(k):
        s = k % 2
        return [
            pltpu.async_copy(sx.at[idxb[s]], gx[s], semG[s]),
            pltpu.async_copy(sy.at[idxb[s]], gy[s], semG[s]),
            pltpu.async_copy(sz.at[idxb[s]], gz[s], semG[s]),
        ]

    def issue_out(k):
        s = k % 2
        sl = pl.ds(wbase + k * CHUNK, CHUNK)
        return [
            pltpu.async_copy(od[s], dist_hbm.at[sl], semO[s]),
            pltpu.async_copy(ovx[s], vx_hbm.at[sl], semO[s]),
            pltpu.async_copy(ovy[s], vy_hbm.at[sl], semO[s]),
            pltpu.async_copy(ovz[s], vz_hbm.at[sl], semO[s]),
        ]

    def compute(k):
        s = k % 2
        cgx, cgy, cgz = gx[s], gy[s], gz[s]
        csh = shb[s]
        cod, cvx, cvy, cvz = od[s], ovx[s], ovy[s], ovz[s]
        cidx = idxb[s]
        a0 = wbase + k * CHUNK - (slice_start(k) << 7)

        @plsc.parallel_loop(0, CHUNK // LANES, 1, unroll=2)
        def vec_body(j):
            e16 = j * LANES
            v = pl.ds(e16, LANES)
            e = a0 + e16
            blk = e >> 7
            ro = e + (blk << 7)
            so = e + (blk << 8)
            vr = pl.ds(ro, LANES)
            vc = pl.ds(ro + 128, LANES)
            m = cidx[vr] != cidx[vc]
            d0 = cgx[vr] - cgx[vc] - csh[pl.ds(so, LANES)]
            d1 = cgy[vr] - cgy[vc] - csh[pl.ds(so + 128, LANES)]
            d2 = cgz[vr] - cgz[vc] - csh[pl.ds(so + 256, LANES)]
            sq = d0 * d0 + d1 * d1 + d2 * d2
            bits = plsc.bitcast(sq, jnp.int32)
            y = plsc.bitcast(jnp.int32(0x5F3759DF) - (bits >> 1), jnp.float32)
            for _ in range(3):
                y = y * (1.5 - 0.5 * sq * y * y)
            cod[v] = jnp.where(m, sq * y, 0.0)
            ym = jnp.where(m, y, 1.0)
            cvx[v] = d0 * ym
            cvy[v] = d1 * ym
            cvz[v] = d2 * ym

    hL, hG, hO = {}, {}, {}
    hL[0] = issue_linear(0)
    for h in hL[0]:
        h.wait()
    hG[0] = issue_gathers(0)
    if n_chunks > 1:
        hL[1] = issue_linear(1)
    for k in range(n_chunks):
        if k + 1 < n_chunks:
            for h in hL[k + 1]:
                h.wait()
            hG[k + 1] = issue_gathers(k + 1)
        for h in hG[k]:
            h.wait()
        if k >= 2:
            for h in hO[k - 2]:
                h.wait()
        compute(k)
        hO[k] = issue_out(k)
        if k + 2 < n_chunks:
            hL[k + 2] = issue_linear(k + 2)
    for k in (n_chunks - 2, n_chunks - 1):
        for h in hO[k]:
            h.wait()


def kernel(positions, edge_indices, shift):
    px = jnp.asarray(positions[:, 0])
    py = jnp.asarray(positions[:, 1])
    pz = jnp.asarray(positions[:, 2])
    n_blocks = edge_indices.shape[1] // 128
    el = edge_indices.reshape(2, n_blocks, 128).transpose(1, 0, 2).reshape(-1)
    sh3 = shift.reshape(n_blocks, 128, 3).transpose(0, 2, 1).reshape(-1)
    n_edges = edge_indices.shape[1]
    mesh = plsc.VectorSubcoreMesh(core_axis_name="c", subcore_axis_name="s")
    f = pltpu.VMEM((CHUNK,), jnp.float32)
    f2 = pltpu.VMEM((SLICE_W,), jnp.float32)
    i2 = pltpu.VMEM((SLICE_W,), jnp.int32)
    fn = pl.kernel(
        _edge_body,
        out_type=(jax.ShapeDtypeStruct((n_edges,), jnp.float32),) * 4,
        mesh=mesh,
        compiler_params=pltpu.CompilerParams(needs_layout_passes=False),
        scratch_types=[
            i2, i2,
            f2, f2, f2, f2, f2, f2,
            pltpu.VMEM((SLICE_B * 384,), jnp.float32),
            pltpu.VMEM((SLICE_B * 384,), jnp.float32),
            f, f, f, f, f, f, f, f,
            pltpu.VMEM_SHARED((N_NODES,), jnp.float32),
            pltpu.VMEM_SHARED((N_NODES,), jnp.float32),
            pltpu.VMEM_SHARED((N_NODES,), jnp.float32),
            pltpu.SemaphoreType.DMA,
            pltpu.SemaphoreType.DMA,
            pltpu.SemaphoreType.DMA,
            pltpu.SemaphoreType.DMA,
            pltpu.SemaphoreType.DMA,
        ],
    )
    dist, vx, vy, vz = fn(px, py, pz, el, sh3)
    vec = jnp.stack([vx, vy, vz], axis=0).T
    return (edge_indices, dist, vec)


# speedup vs baseline: 1.6569x; 1.3297x over previous
import jax
import jax.numpy as jnp
from jax import lax
from jax.experimental import pallas as pl
from jax.experimental.pallas import tpu as pltpu
from jax.experimental.pallas import tpu_sc as plsc

N_CORES = 2
N_SUBCORES = 16
LANES = 16
N_WORKERS = N_CORES * N_SUBCORES

CB = 15
CHUNK = CB * 128
WORKER_BLOCKS = 390
N_CHUNKS = WORKER_BLOCKS // CB
IDX_W = CB * 256
SH_W = CB * 384
N_NODES = 50000


def _edge_body(px, py, pz, el_hbm, sh3_hbm,
               dist_hbm, v3_hbm,
               idxb0, idxb1, gx0, gx1, gy0, gy1, gz0, gz1,
               shb0, shb1, od0, od1, ovb0, ovb1,
               sx, sy, sz, semL, semG0, semG1, semO0, semO1):
    n_blocks = dist_hbm.shape[0] // 128
    sid = lax.axis_index("s")
    wid = sid * N_CORES + lax.axis_index("c")
    wb = wid * WORKER_BLOCKS

    idxb = (idxb0, idxb1)
    gx, gy, gz = (gx0, gx1), (gy0, gy1), (gz0, gz1)
    shb = (shb0, shb1)
    od, ovb = (od0, od1), (ovb0, ovb1)
    semG = (semG0, semG1)
    semO = (semO0, semO1)

    @pl.when(sid == 0)
    def _stage():
        pltpu.sync_copy(px, sx)
        pltpu.sync_copy(py, sy)
        pltpu.sync_copy(pz, sz)

    plsc.subcore_barrier()

    def issue_linear(k):
        s = k % 2
        b0 = wb + k * CB
        return [
            pltpu.async_copy(el_hbm.at[pl.ds(b0 * 256, IDX_W)], idxb[s], semL),
            pltpu.async_copy(sh3_hbm.at[pl.ds(b0 * 384, SH_W)], shb[s], semL),
        ]

    def issue_gathers(k):
        s = k % 2
        return [
            pltpu.async_copy(sx.at[idxb[s]], gx[s], semG[s]),
            pltpu.async_copy(sy.at[idxb[s]], gy[s], semG[s]),
            pltpu.async_copy(sz.at[idxb[s]], gz[s], semG[s]),
        ]

    def issue_out(k):
        s = k % 2
        b0 = wb + k * CB
        return [
            pltpu.async_copy(od[s], dist_hbm.at[pl.ds(b0 * 128, CHUNK)], semO[s]),
            pltpu.async_copy(ovb[s], v3_hbm.at[pl.ds(b0 * 384, SH_W)], semO[s]),
        ]

    def body16(cidx, cgx, cgy, cgz, csh, cod, cvb, j):
        e16 = j * LANES
        v = pl.ds(e16, LANES)
        blk = j // 8
        ro = e16 + (blk << 7)
        so = e16 + (blk << 8)
        vr = pl.ds(ro, LANES)
        vc = pl.ds(ro + 128, LANES)
        m = cidx[vr] != cidx[vc]
        d0 = cgx[vr] - cgx[vc] - csh[pl.ds(so, LANES)]
        d1 = cgy[vr] - cgy[vc] - csh[pl.ds(so + 128, LANES)]
        d2 = cgz[vr] - cgz[vc] - csh[pl.ds(so + 256, LANES)]
        sq = d0 * d0 + d1 * d1 + d2 * d2
        bits = plsc.bitcast(sq, jnp.int32)
        y = plsc.bitcast(jnp.int32(0x5F3759DF) - (bits >> 1), jnp.float32)
        for _ in range(3):
            y = y * (1.5 - 0.5 * sq * y * y)
        cod[v] = jnp.where(m, sq * y, 0.0)
        ym = jnp.where(m, y, 1.0)
        cvb[pl.ds(so, LANES)] = d0 * ym
        cvb[pl.ds(so + 128, LANES)] = d1 * ym
        cvb[pl.ds(so + 256, LANES)] = d2 * ym

    def compute(k, n16):
        s = k % 2
        args = (idxb[s], gx[s], gy[s], gz[s], shb[s], od[s], ovb[s])

        @plsc.parallel_loop(0, n16, 1, unroll=2)
        def vec_body(j):
            body16(*args, j)

    hL, hG, hO = {}, {}, {}
    hL[0] = issue_linear(0)
    for h in hL[0]:
        h.wait()
    hG[0] = issue_gathers(0)
    hL[1] = issue_linear(1)
    for k in range(N_CHUNKS):
        if k + 1 < N_CHUNKS:
            for h in hL[k + 1]:
                h.wait()
            hG[k + 1] = issue_gathers(k + 1)
        for h in hG[k]:
            h.wait()
        if k >= 2:
            for h in hO[k - 2]:
                h.wait()
        compute(k, CHUNK // LANES)
        hO[k] = issue_out(k)
        if k + 2 < N_CHUNKS:
            hL[k + 2] = issue_linear(k + 2)
    for k in (N_CHUNKS - 2, N_CHUNKS - 1):
        for h in hO[k]:
            h.wait()

    tail_blocks = n_blocks - N_WORKERS * WORKER_BLOCKS

    @pl.when(wid < tail_blocks)
    def _tail():
        tb = N_WORKERS * WORKER_BLOCKS + wid
        pltpu.sync_copy(el_hbm.at[pl.ds(tb * 256, 256)],
                        idxb0.at[pl.ds(0, 256)])
        pltpu.sync_copy(sh3_hbm.at[pl.ds(tb * 384, 384)],
                        shb0.at[pl.ds(0, 384)])
        cps = [pltpu.async_copy(sx.at[idxb0.at[pl.ds(0, 256)]],
                                gx0.at[pl.ds(0, 256)], semG0),
               pltpu.async_copy(sy.at[idxb0.at[pl.ds(0, 256)]],
                                gy0.at[pl.ds(0, 256)], semG0),
               pltpu.async_copy(sz.at[idxb0.at[pl.ds(0, 256)]],
                                gz0.at[pl.ds(0, 256)], semG0)]
        for cp in cps:
            cp.wait()

        @plsc.parallel_loop(0, 128 // LANES, 1, unroll=2)
        def tail_body(j):
            body16(idxb0, gx0, gy0, gz0, shb0, od0, ovb0, j)

        pltpu.sync_copy(od0.at[pl.ds(0, 128)],
                        dist_hbm.at[pl.ds(tb * 128, 128)])
        pltpu.sync_copy(ovb0.at[pl.ds(0, 384)],
                        v3_hbm.at[pl.ds(tb * 384, 384)])


def kernel(positions, edge_indices, shift):
    px = jnp.asarray(positions[:, 0])
    py = jnp.asarray(positions[:, 1])
    pz = jnp.asarray(positions[:, 2])
    n_edges = edge_indices.shape[1]
    n_blocks = n_edges // 128
    el = edge_indices.reshape(2, n_blocks, 128).transpose(1, 0, 2).reshape(-1)
    sh3 = shift.reshape(n_blocks, 128, 3).transpose(0, 2, 1).reshape(-1)
    mesh = plsc.VectorSubcoreMesh(core_axis_name="c", subcore_axis_name="s")
    fo = pltpu.VMEM((CHUNK,), jnp.float32)
    fg = pltpu.VMEM((IDX_W,), jnp.float32)
    fs = pltpu.VMEM((SH_W,), jnp.float32)
    ig = pltpu.VMEM((IDX_W,), jnp.int32)
    fn = pl.kernel(
        _edge_body,
        out_type=(jax.ShapeDtypeStruct((n_edges,), jnp.float32),
                  jax.ShapeDtypeStruct((n_blocks * 384,), jnp.float32)),
        mesh=mesh,
        compiler_params=pltpu.CompilerParams(needs_layout_passes=False),
        scratch_types=[
            ig, ig,
            fg, fg, fg, fg, fg, fg,
            fs, fs,
            fo, fo,
            fs, fs,
            pltpu.VMEM_SHARED((N_NODES,), jnp.float32),
            pltpu.VMEM_SHARED((N_NODES,), jnp.float32),
            pltpu.VMEM_SHARED((N_NODES,), jnp.float32),
            pltpu.SemaphoreType.DMA,
            pltpu.SemaphoreType.DMA,
            pltpu.SemaphoreType.DMA,
            pltpu.SemaphoreType.DMA,
            pltpu.SemaphoreType.DMA,
        ],
    )
    dist, v3 = fn(px, py, pz, el, sh3)
    vec = v3.reshape(n_blocks, 3, 128).transpose(0, 2, 1).reshape(n_edges, 3)
    return (edge_indices, dist, vec)
